# baseline (speedup 1.0000x reference)
import functools

import jax
import jax.numpy as jnp
from jax import lax
from jax.experimental import pallas as pl
from jax.experimental.pallas import tpu as pltpu
from jax.experimental.pallas import tpu_sc as plsc

Q_LEN = 1
L = 32
BSZ = 64
NHID = 256
N = 32
DK = L * NHID
TOPK = 8
LANES = 16
NC = 2
NS = 16
NW = NC * NS
SCALE = 1.0 / float(DK) ** 0.5

B_TC = 32
B_SC = BSZ - B_TC

SG = 8
DH = 4096
NH = DK // DH
NGROUPS = N // SG


def _round_bf16_pair(x, y):
    pu = plsc.bitcast(plsc.pack(x, y, format=plsc.PackFormat.INTERLEAVED),
                      jnp.uint32)
    xr = plsc.bitcast(pu << 16, jnp.float32)
    yr = plsc.bitcast(pu & jnp.uint32(0xFFFF0000), jnp.float32)
    return xr, yr


def _sc_body(q_hbm, keys_hbm, attn_hbm, topk_hbm, qv, kb0, kb1, sv, tv,
             sem_q, sem0, sem1):
    wid = lax.axis_index("s") * NC + lax.axis_index("c")
    idx0 = lax.iota(jnp.int32, LANES)
    idx1 = idx0 + LANES
    kbufs = (kb0, kb1)
    sems = (sem0, sem1)

    b = B_TC + wid
    qcps = [pltpu.async_copy(q_hbm.at[0, l, b, :],
                             qv.at[pl.ds(l * NHID, NHID)], sem_q)
            for l in range(L)]

    def fire(g):
        sg, h = divmod(g, NH)
        buf = kbufs[g % 2]
        sem = sems[g % 2]
        return [pltpu.async_copy(
            keys_hbm.at[sg * SG + s, b, pl.ds(h * DH, DH)],
            buf.at[s], sem) for s in range(SG)]

    cps = {0: fire(0)}

    for cp in qcps:
        cp.wait()

    @plsc.parallel_loop(0, DK // (2 * LANES), unroll=4)
    def q_round_body(i):
        q0, q1 = _round_bf16_pair(qv[pl.ds(i * 2 * LANES, LANES)],
                                  qv[pl.ds(i * 2 * LANES + LANES, LANES)])
        qv[pl.ds(i * 2 * LANES, LANES)] = q0
        qv[pl.ds(i * 2 * LANES + LANES, LANES)] = q1

    s0 = jnp.zeros((LANES,), jnp.float32)
    s1 = jnp.zeros((LANES,), jnp.float32)
    accs = None
    for g in range(NH * NGROUPS):
        sg, h = divmod(g, NH)
        if g + 1 < NH * NGROUPS:
            cps[g + 1] = fire(g + 1)
        for cp in cps.pop(g):
            cp.wait()
        buf = kbufs[g % 2]
        if h == 0:
            accs = (jnp.zeros((LANES,), jnp.float32),) * SG
        qoff = h * DH

        def dot_body(i, accs):
            q0 = qv[pl.ds(qoff + i * 2 * LANES, LANES)]
            q1 = qv[pl.ds(qoff + i * 2 * LANES + LANES, LANES)]
            out = []
            for s in range(SG):
                k0, k1 = _round_bf16_pair(
                    buf[s, pl.ds(i * 2 * LANES, LANES)],
                    buf[s, pl.ds(i * 2 * LANES + LANES, LANES)])
                out.append(accs[s] + k0 * q0 + k1 * q1)
            return tuple(out)

        accs = plsc.parallel_loop(0, DH // (2 * LANES), unroll=4,
                                  carry=accs)(dot_body)
        if h == NH - 1:
            for s in range(SG):
                n = sg * SG + s
                score = jnp.sum(accs[s]) * SCALE
                if n < LANES:
                    s0 = jnp.where(idx0 == n, score, s0)
                else:
                    s1 = jnp.where(idx0 == (n - LANES), score, s1)

    m = jnp.maximum(jnp.max(s0), jnp.max(s1))
    e0 = jnp.exp(s0 - m)
    e1 = jnp.exp(s1 - m)
    denom = jnp.sum(e0) + jnp.sum(e1)
    a0 = e0 / denom
    a1 = e1 / denom

    sv[pl.ds(0, LANES)] = a0
    sv[pl.ds(LANES, LANES)] = a1
    pltpu.sync_copy(sv, attn_hbm.at[pl.ds(wid * N, N)])

    r0 = jnp.zeros((LANES,), jnp.int32)
    r1 = jnp.zeros((LANES,), jnp.int32)
    for mi in range(N):
        am_s = a0[mi] if mi < LANES else a1[mi - LANES]
        am = jnp.broadcast_to(am_s, (LANES,))
        r0 = r0 + (am > a0).astype(jnp.int32)
        r1 = r1 + (am > a1).astype(jnp.int32)
        r0 = r0 + ((am == a0) & (idx0 > mi)).astype(jnp.int32)
        r1 = r1 + ((am == a1) & (idx1 > mi)).astype(jnp.int32)

    plsc.store_scatter(tv, [r0], idx0, mask=r0 < TOPK)
    plsc.store_scatter(tv, [r1], idx1, mask=r1 < TOPK)
    pltpu.sync_copy(tv.at[pl.ds(0, TOPK)], topk_hbm.at[pl.ds(wid * TOPK, TOPK)])


@functools.partial(
    pl.kernel,
    mesh=plsc.VectorSubcoreMesh(core_axis_name="c", subcore_axis_name="s"),
    out_type=[
        jax.ShapeDtypeStruct((B_SC * N,), jnp.float32),
        jax.ShapeDtypeStruct((B_SC * TOPK,), jnp.int32),
    ],
    scratch_types=[
        pltpu.VMEM((DK,), jnp.float32),
        pltpu.VMEM((SG, DH), jnp.float32),
        pltpu.VMEM((SG, DH), jnp.float32),
        pltpu.VMEM((N,), jnp.float32),
        pltpu.VMEM((LANES,), jnp.int32),
        pltpu.SemaphoreType.DMA,
        pltpu.SemaphoreType.DMA,
        pltpu.SemaphoreType.DMA,
    ],
    compiler_params=pltpu.CompilerParams(needs_layout_passes=False),
)
def _sc_half(q_hbm, keys_hbm, attn_hbm, topk_hbm, qv, kb0, kb1, sv, tv,
             sem_q, sem0, sem1):
    _sc_body(q_hbm, keys_hbm, attn_hbm, topk_hbm, qv, kb0, kb1, sv, tv,
             sem_q, sem0, sem1)


NBLK = 2


def _tc_body(keys_ref, q_ref, attn_ref, topk_ref, scores_ref):
    n = pl.program_id(0)
    kr = keys_ref[...].astype(jnp.bfloat16).astype(jnp.float32)
    qr = q_ref[...].astype(jnp.float32)
    partial = jnp.sum(kr * qr[None], axis=2) * SCALE
    scores_ref[pl.ds(n * NBLK, NBLK), :] = partial

    @pl.when(n == N // NBLK - 1)
    def _():
        s = scores_ref[...]
        m = jnp.max(s, axis=0, keepdims=True)
        e = jnp.exp(s - m)
        att_t = e / jnp.sum(e, axis=0, keepdims=True)
        attn_ref[...] = jnp.transpose(att_t)[:, None, :]

        srow = jax.lax.broadcasted_iota(jnp.int32, (N, B_TC), 0)
        rank = jnp.zeros((N, B_TC), jnp.int32)
        for mi in range(N):
            am = att_t[mi:mi + 1, :]
            rank = rank + (am > att_t).astype(jnp.int32)
            rank = rank + ((am == att_t) & (srow > mi)).astype(jnp.int32)
        for k in range(TOPK):
            topk_ref[k, :] = jnp.sum(jnp.where(rank == k, srow, 0), axis=0)


_tc_call = pl.pallas_call(
    _tc_body,
    grid=(N // NBLK,),
    in_specs=[
        pl.BlockSpec((NBLK, B_TC, DK), lambda n: (n, 0, 0)),
        pl.BlockSpec((B_TC, DK), lambda n: (0, 0)),
    ],
    out_specs=[
        pl.BlockSpec((B_TC, 1, N), lambda n: (0, 0, 0)),
        pl.BlockSpec((TOPK, B_TC), lambda n: (0, 0)),
    ],
    out_shape=[
        jax.ShapeDtypeStruct((B_TC, 1, N), jnp.float32),
        jax.ShapeDtypeStruct((TOPK, B_TC), jnp.int32),
    ],
    scratch_shapes=[pltpu.VMEM((N, B_TC), jnp.float32)],
)


def kernel(query, keys, values):
    del values
    attn_sc_flat, topk_sc_flat = _sc_half(query, keys)
    qf = jnp.transpose(query[:, :, :B_TC, :], (0, 2, 1, 3)).reshape(B_TC, DK)
    attn_tc, topk_tc = _tc_call(keys, qf.astype(jnp.bfloat16))
    attention = jnp.concatenate(
        [attn_tc, attn_sc_flat.reshape(B_SC, 1, N)], axis=0)
    topk_indices = jnp.concatenate(
        [topk_tc, topk_sc_flat.reshape(B_SC, TOPK).T], axis=1)
    return attention, topk_indices
